# Initial kernel scaffold; baseline (speedup 1.0000x reference)
import functools

import jax
import jax.numpy as jnp
from jax import lax
from jax.experimental import pallas as pl
from jax.experimental.pallas import tpu as pltpu
from jax.experimental.pallas import tpu_sc as plsc

N = 10000
E = 320000
D = 128
NC = 2
NS = 16
NW = NC * NS
EP = E // NW
CH = 80
NCH = EP // CH
RPT = N // NS
ZR = 125

_mesh = plsc.VectorSubcoreMesh(core_axis_name="c", subcore_axis_name="s")


@functools.partial(
    pl.kernel,
    out_type=jax.ShapeDtypeStruct((NC, N, D), jnp.float32),
    mesh=_mesh,
    scratch_types=[
        pltpu.VMEM((CH,), jnp.int32),
        pltpu.VMEM((CH,), jnp.int32),
        pltpu.VMEM((CH,), jnp.float32),
        pltpu.VMEM((CH, D), jnp.float32),
        pltpu.VMEM((ZR, D), jnp.float32),
        pltpu.VMEM_SHARED((N, D), jnp.float32),
        pltpu.SemaphoreType.DMA,
    ],
)
def _propagate(x_hbm, src_hbm, dst_hbm, w_hbm, out_hbm,
               src_v, dst_v, w_v, rows_v, zero_v, acc_sh, sem):
    cid = lax.axis_index("c")
    sid = lax.axis_index("s")
    wid = cid * NS + sid

    zeros16 = jnp.zeros((16,), jnp.float32)

    def zfill(i, carry):
        for cc in range(D // 16):
            zero_v[i, pl.ds(cc * 16, 16)] = zeros16
        return carry

    lax.fori_loop(0, ZR, zfill, 0)
    for j in range(RPT // ZR):
        pltpu.sync_copy(zero_v, acc_sh.at[pl.ds(sid * RPT + j * ZR, ZR)])
    plsc.subcore_barrier()

    def chunk_body(c, carry):
        base = wid * EP + c * CH
        pltpu.sync_copy(src_hbm.at[pl.ds(base, CH)], src_v)
        pltpu.sync_copy(dst_hbm.at[pl.ds(base, CH)], dst_v)
        pltpu.sync_copy(w_hbm.at[pl.ds(base, CH)], w_v)
        pltpu.async_copy(x_hbm.at[src_v], rows_v, sem).wait()

        def edge_body(e, ecarry):
            wspl = plsc.load_gather(w_v, [jnp.full((16,), e, jnp.int32)])
            for cc in range(D // 16):
                sl = pl.ds(cc * 16, 16)
                rows_v[e, sl] = rows_v[e, sl] * wspl
            return ecarry

        lax.fori_loop(0, CH, edge_body, 0)
        pltpu.sync_copy(rows_v, acc_sh.at[dst_v], add=True)
        return carry

    lax.fori_loop(0, NCH, chunk_body, 0)
    plsc.subcore_barrier()

    for j in range(RPT // ZR):
        r0 = sid * RPT + j * ZR
        pltpu.sync_copy(acc_sh.at[pl.ds(r0, ZR)], out_hbm.at[cid, pl.ds(r0, ZR)])


_BM = 2000


def _mm_body(hp_ref, w_ref, o_ref):
    h = hp_ref[0] + hp_ref[1]
    o_ref[...] = jnp.dot(h, w_ref[...], preferred_element_type=jnp.float32)


def _matmul(hp, W0):
    return pl.pallas_call(
        _mm_body,
        grid=(N // _BM,),
        in_specs=[
            pl.BlockSpec((NC, _BM, D), lambda i: (0, i, 0)),
            pl.BlockSpec((D, D), lambda i: (0, 0)),
        ],
        out_specs=pl.BlockSpec((_BM, D), lambda i: (i, 0)),
        out_shape=jax.ShapeDtypeStruct((N, D), jnp.float32),
    )(hp, W0)


def kernel(x, edge_index, edge_weight, W0):
    dst = edge_index[0].astype(jnp.int32)
    src = edge_index[1].astype(jnp.int32)
    hp = _propagate(x, src, dst, edge_weight)
    return _matmul(hp, W0)


# speedup vs baseline: 4.1691x; 4.1691x over previous
import functools

import jax
import jax.numpy as jnp
from jax import lax
from jax.experimental import pallas as pl
from jax.experimental.pallas import tpu as pltpu
from jax.experimental.pallas import tpu_sc as plsc

N = 10000
E = 320000
D = 128
NC = 2
NS = 16
NW = NC * NS
EP = E // NW
CH = 80
NCH = EP // CH
ZR = 40
NZC = N // ZR

_mesh = plsc.VectorSubcoreMesh(core_axis_name="c", subcore_axis_name="s")


@functools.partial(
    pl.kernel,
    out_type=jax.ShapeDtypeStruct((NC, N, D), jnp.float32),
    mesh=_mesh,
    scratch_types=[
        pltpu.VMEM((CH,), jnp.int32),
        pltpu.VMEM((CH,), jnp.int32),
        pltpu.VMEM((CH,), jnp.float32),
        pltpu.VMEM((CH, D), jnp.float32),
        pltpu.VMEM((ZR, D), jnp.float32),
        pltpu.VMEM_SHARED((N, D), jnp.float32),
        pltpu.SemaphoreType.DMA,
    ],
)
def _propagate(x_hbm, src_hbm, dst_hbm, w_hbm, out_hbm,
               src_v, dst_v, w_v, rows_v, zero_v, acc_sh, sem):
    cid = lax.axis_index("c")
    sid = lax.axis_index("s")
    wid = cid * NS + sid

    zeros16 = jnp.zeros((16,), jnp.float32)
    my_chunks = jnp.where(sid < NZC % NS, NZC // NS + 1, NZC // NS)

    def zfill(i, carry):
        for cc in range(D // 16):
            zero_v[i, pl.ds(cc * 16, 16)] = zeros16
        return carry

    lax.fori_loop(0, ZR, zfill, 0)

    def zcopy(k, carry):
        r0 = pl.multiple_of((sid + k * NS) * ZR, 8)
        pltpu.sync_copy(zero_v, acc_sh.at[pl.ds(r0, ZR)])
        return carry

    lax.fori_loop(0, my_chunks, zcopy, 0)
    plsc.subcore_barrier()

    def chunk_body(c, carry):
        base = pl.multiple_of(wid * EP + c * CH, 8)
        pltpu.sync_copy(src_hbm.at[pl.ds(base, CH)], src_v)
        pltpu.sync_copy(dst_hbm.at[pl.ds(base, CH)], dst_v)
        pltpu.sync_copy(w_hbm.at[pl.ds(base, CH)], w_v)
        pltpu.async_copy(x_hbm.at[src_v], rows_v, sem).wait()

        def group_body(g, gcarry):
            w16 = w_v[pl.ds(g * 16, 16)]
            for j in range(16):
                wspl = lax.gather(
                    w16, jnp.full((16, 1), j, jnp.int32),
                    dimension_numbers=lax.GatherDimensionNumbers(
                        offset_dims=(), collapsed_slice_dims=(0,),
                        start_index_map=(0,)),
                    slice_sizes=(1,),
                    mode=lax.GatherScatterMode.PROMISE_IN_BOUNDS)
                e = g * 16 + j
                for cc in range(D // 16):
                    sl = pl.ds(cc * 16, 16)
                    rows_v[e, sl] = rows_v[e, sl] * wspl
            return gcarry

        lax.fori_loop(0, CH // 16, group_body, 0)
        pltpu.sync_copy(rows_v, acc_sh.at[dst_v], add=True)
        return carry

    lax.fori_loop(0, NCH, chunk_body, 0)
    plsc.subcore_barrier()

    def ocopy(k, carry):
        r0 = pl.multiple_of((sid + k * NS) * ZR, 8)
        pltpu.sync_copy(acc_sh.at[pl.ds(r0, ZR)],
                        out_hbm.at[cid, pl.ds(r0, ZR)])
        return carry

    lax.fori_loop(0, my_chunks, ocopy, 0)


_BM = 2000


def _mm_body(hp_ref, w_ref, o_ref):
    h = hp_ref[0] + hp_ref[1]
    o_ref[...] = jnp.dot(h, w_ref[...], preferred_element_type=jnp.float32)


def _matmul(hp, W0):
    return pl.pallas_call(
        _mm_body,
        grid=(N // _BM,),
        in_specs=[
            pl.BlockSpec((NC, _BM, D), lambda i: (0, i, 0)),
            pl.BlockSpec((D, D), lambda i: (0, 0)),
        ],
        out_specs=pl.BlockSpec((_BM, D), lambda i: (i, 0)),
        out_shape=jax.ShapeDtypeStruct((N, D), jnp.float32),
    )(hp, W0)


def kernel(x, edge_index, edge_weight, W0):
    dst = edge_index[0].astype(jnp.int32)
    src = edge_index[1].astype(jnp.int32)
    hp = _propagate(x, src, dst, edge_weight)
    return _matmul(hp, W0)
